# baseline (speedup 1.0000x reference)
import functools

import jax
import jax.numpy as jnp
from jax import lax
from jax.experimental import pallas as pl
from jax.experimental.pallas import tpu as pltpu
from jax.experimental.pallas import tpu_sc as plsc

L = 64
N = 16384
D = 16
B = 8

NC = 2
NS = 16
NW = NC * NS
CHUNK = N // NW
LANES = 16
GPR = CHUNK // LANES
LBITS = 6


def _body(histT_hbm, valueT_hbm, pack_hbm, outT_hbm,
          pack_v, d00, d01, d10, d11, ob0, ob1,
          sg00, sg01, sg10, sg11, so0, so1):
    wid = lax.axis_index("s") * NC + lax.axis_index("c")
    ns = pl.ds(wid * CHUNK, CHUNK)

    pltpu.sync_copy(pack_hbm, pack_v)

    idxv = plsc.bitcast(pack_v[2, pl.ds(0, LANES)], jnp.int32)
    selv = plsc.bitcast(pack_v[3, pl.ds(0, LANES)], jnp.int32)
    lane = lax.broadcasted_iota(jnp.int32, (LANES,), 0)

    def extract(vec, m):
        return jnp.max(jnp.where(m, vec, 0))

    i0s, i1s, s0s, s1s = {}, {}, {}, {}

    def extract_entry(b):
        m0 = lane == b
        m1 = lane == (B + b)
        i0s[b] = extract(idxv, m0)
        i1s[b] = extract(idxv, m1)
        s0s[b] = jnp.any(m0 & (selv == 1))
        s1s[b] = jnp.any(m1 & (selv == 1))

    d0bufs = (d00, d01)
    d1bufs = (d10, d11)
    obufs = (ob0, ob1)
    g0sems = (sg00, sg01)
    g1sems = (sg10, sg11)
    osems = (so0, so1)

    def issue(b):
        p = b % 2
        d0, d1 = d0bufs[p], d1bufs[p]

        @pl.when(s0s[b])
        def _():
            pltpu.async_copy(valueT_hbm.at[:, ns], d0, g0sems[p])

        @pl.when(jnp.logical_not(s0s[b]))
        def _():
            pltpu.async_copy(histT_hbm.at[i0s[b], :, ns], d0, g0sems[p])

        @pl.when(s1s[b])
        def _():
            pltpu.async_copy(valueT_hbm.at[:, ns], d1, g1sems[p])

        @pl.when(jnp.logical_not(s1s[b]))
        def _():
            pltpu.async_copy(histT_hbm.at[i1s[b], :, ns], d1, g1sems[p])

    extract_entry(0)
    issue(0)
    for b in range(1, B):
        extract_entry(b)
    out_pending = [None, None]

    for b in range(B):
        p = b % 2
        if b < B - 1:
            issue(b + 1)
        pltpu.make_async_copy(histT_hbm.at[0, :, ns], d0bufs[p], g0sems[p]).wait()
        pltpu.make_async_copy(histT_hbm.at[0, :, ns], d1bufs[p], g1sems[p]).wait()
        if out_pending[p] is not None:
            out_pending[p].wait()
            out_pending[p] = None

        d0, d1, ob = d0bufs[p], d1bufs[p], obufs[p]
        bs = pl.ds(b * LANES, LANES)
        wa = pack_v[0, bs]
        wb = pack_v[1, bs]

        def compute(r, carry):
            for g in range(GPR):
                sl = pl.ds(g * LANES, LANES)
                a = d0[r, sl]
                c = d1[r, sl]
                ob[r, sl] = wa * a + wb * c
            return carry

        lax.fori_loop(0, D, compute, 0, unroll=2)

        out_pending[p] = pltpu.async_copy(ob, outT_hbm.at[b, :, ns], osems[p])

    for p in range(2):
        if out_pending[p] is not None:
            out_pending[p].wait()


@jax.jit
def _sc_delay(histT, valueT, pack):
    call = functools.partial(
        pl.kernel,
        mesh=plsc.VectorSubcoreMesh(core_axis_name="c", subcore_axis_name="s"),
        compiler_params=pltpu.CompilerParams(needs_layout_passes=False),
        out_type=jax.ShapeDtypeStruct((B, D, N), jnp.float32),
        scratch_types=[
            pltpu.VMEM((8, 128), jnp.float32),
            pltpu.VMEM((D, CHUNK), jnp.float32),
            pltpu.VMEM((D, CHUNK), jnp.float32),
            pltpu.VMEM((D, CHUNK), jnp.float32),
            pltpu.VMEM((D, CHUNK), jnp.float32),
            pltpu.VMEM((D, CHUNK), jnp.float32),
            pltpu.VMEM((D, CHUNK), jnp.float32),
            pltpu.SemaphoreType.DMA,
            pltpu.SemaphoreType.DMA,
            pltpu.SemaphoreType.DMA,
            pltpu.SemaphoreType.DMA,
            pltpu.SemaphoreType.DMA,
            pltpu.SemaphoreType.DMA,
        ],
    )(_body)
    return call(histT, valueT, pack)


def kernel(history, value, delay_frac, write_idx, delay_int):
    histT = history.transpose(0, 2, 1)
    valueT = value.transpose(1, 0)

    wi = jnp.asarray(write_idx, jnp.int32)
    i0 = jnp.mod(wi - delay_int, L)
    i1 = jnp.mod(wi - delay_int - 1, L)
    wrow = jnp.mod(wi, L)
    w = delay_frac.astype(jnp.float32)
    ids = jnp.concatenate([i0, i1]).astype(jnp.int32)
    sel = jnp.concatenate([i0 == wrow, i1 == wrow]).astype(jnp.int32)

    pad = jnp.zeros((112,), jnp.float32)
    zrow = jnp.zeros((128,), jnp.float32)
    pack = jnp.stack([
        jnp.repeat(1.0 - w, LANES),
        jnp.repeat(w, LANES),
        jnp.concatenate([jax.lax.bitcast_convert_type(ids, jnp.float32), pad]),
        jnp.concatenate([jax.lax.bitcast_convert_type(sel, jnp.float32), pad]),
        zrow, zrow, zrow, zrow,
    ])

    outT = _sc_delay(histT, valueT, pack)
    return outT.transpose(0, 2, 1)


# speedup vs baseline: 1.0838x; 1.0838x over previous
import functools

import jax
import jax.numpy as jnp
from jax import lax
from jax.experimental import pallas as pl
from jax.experimental.pallas import tpu as pltpu
from jax.experimental.pallas import tpu_sc as plsc

L = 64
N = 16384
D = 16
B = 8

NC = 2
NS = 16
NW = NC * NS
CHUNK = N // NW
LANES = 16
GPR = CHUNK // LANES
LBITS = 6


def _body(histT_hbm, valueT_hbm, pack_hbm, outT_hbm,
          pack_v, d00, d01, d10, d11, ob0, ob1,
          sg00, sg01, sg10, sg11, so0, so1):
    wid = lax.axis_index("s") * NC + lax.axis_index("c")
    ns = pl.ds(wid * CHUNK, CHUNK)

    pltpu.sync_copy(pack_hbm, pack_v)

    idxv = plsc.bitcast(pack_v[2, pl.ds(0, LANES)], jnp.int32)
    selv = plsc.bitcast(pack_v[3, pl.ds(0, LANES)], jnp.int32)
    lane = lax.broadcasted_iota(jnp.int32, (LANES,), 0)

    def extract(vec, m):
        return jnp.max(jnp.where(m, vec, 0))

    i0s, i1s, s0s, s1s = {}, {}, {}, {}

    def extract_entry(b):
        m0 = lane == b
        m1 = lane == (B + b)
        i0s[b] = extract(idxv, m0)
        i1s[b] = extract(idxv, m1)
        s0s[b] = jnp.any(m0 & (selv == 1))
        s1s[b] = jnp.any(m1 & (selv == 1))

    d0bufs = (d00, d01)
    d1bufs = (d10, d11)
    obufs = (ob0, ob1)
    g0sems = (sg00, sg01)
    g1sems = (sg10, sg11)
    osems = (so0, so1)

    def issue(b):
        p = b % 2
        d0, d1 = d0bufs[p], d1bufs[p]

        @pl.when(s0s[b])
        def _():
            pltpu.async_copy(valueT_hbm.at[:, ns], d0, g0sems[p])

        @pl.when(jnp.logical_not(s0s[b]))
        def _():
            pltpu.async_copy(histT_hbm.at[i0s[b], :, ns], d0, g0sems[p])

        @pl.when(s1s[b])
        def _():
            pltpu.async_copy(valueT_hbm.at[:, ns], d1, g1sems[p])

        @pl.when(jnp.logical_not(s1s[b]))
        def _():
            pltpu.async_copy(histT_hbm.at[i1s[b], :, ns], d1, g1sems[p])

    extract_entry(0)
    issue(0)
    for b in range(1, B):
        extract_entry(b)
    out_pending = [None, None]

    for b in range(B):
        p = b % 2
        if b < B - 1:
            issue(b + 1)
        pltpu.make_async_copy(histT_hbm.at[0, :, ns], d0bufs[p], g0sems[p]).wait()
        pltpu.make_async_copy(histT_hbm.at[0, :, ns], d1bufs[p], g1sems[p]).wait()
        if out_pending[p] is not None:
            out_pending[p].wait()
            out_pending[p] = None

        d0, d1, ob = d0bufs[p], d1bufs[p], obufs[p]
        bs = pl.ds(b * LANES, LANES)
        wa = pack_v[0, bs]
        wb = pack_v[1, bs]

        def compute(r, carry):
            for g in range(GPR):
                sl = pl.ds(g * LANES, LANES)
                a = d0[r, sl]
                c = d1[r, sl]
                ob[r, sl] = wa * a + wb * c
            return carry

        lax.fori_loop(0, D, compute, 0)

        out_pending[p] = pltpu.async_copy(ob, outT_hbm.at[b, :, ns], osems[p])

    for p in range(2):
        if out_pending[p] is not None:
            out_pending[p].wait()


@jax.jit
def _sc_delay(histT, valueT, pack):
    call = functools.partial(
        pl.kernel,
        mesh=plsc.VectorSubcoreMesh(core_axis_name="c", subcore_axis_name="s"),
        compiler_params=pltpu.CompilerParams(needs_layout_passes=False),
        out_type=jax.ShapeDtypeStruct((B, D, N), jnp.float32),
        scratch_types=[
            pltpu.VMEM((8, 128), jnp.float32),
            pltpu.VMEM((D, CHUNK), jnp.float32),
            pltpu.VMEM((D, CHUNK), jnp.float32),
            pltpu.VMEM((D, CHUNK), jnp.float32),
            pltpu.VMEM((D, CHUNK), jnp.float32),
            pltpu.VMEM((D, CHUNK), jnp.float32),
            pltpu.VMEM((D, CHUNK), jnp.float32),
            pltpu.SemaphoreType.DMA,
            pltpu.SemaphoreType.DMA,
            pltpu.SemaphoreType.DMA,
            pltpu.SemaphoreType.DMA,
            pltpu.SemaphoreType.DMA,
            pltpu.SemaphoreType.DMA,
        ],
    )(_body)
    return call(histT, valueT, pack)


def kernel(history, value, delay_frac, write_idx, delay_int):
    histT = history.transpose(0, 2, 1)
    valueT = value.transpose(1, 0)

    wi = jnp.asarray(write_idx, jnp.int32)
    i0 = jnp.mod(wi - delay_int, L)
    i1 = jnp.mod(wi - delay_int - 1, L)
    wrow = jnp.mod(wi, L)
    w = delay_frac.astype(jnp.float32)
    ids = jnp.concatenate([i0, i1]).astype(jnp.int32)
    sel = jnp.concatenate([i0 == wrow, i1 == wrow]).astype(jnp.int32)

    pad = jnp.zeros((112,), jnp.float32)
    zrow = jnp.zeros((128,), jnp.float32)
    pack = jnp.stack([
        jnp.repeat(1.0 - w, LANES),
        jnp.repeat(w, LANES),
        jnp.concatenate([jax.lax.bitcast_convert_type(ids, jnp.float32), pad]),
        jnp.concatenate([jax.lax.bitcast_convert_type(sel, jnp.float32), pad]),
        zrow, zrow, zrow, zrow,
    ])

    outT = _sc_delay(histT, valueT, pack)
    return outT.transpose(0, 2, 1)
